# baseline (speedup 1.0000x reference)
import jax
import jax.numpy as jnp
from jax import lax
from jax.experimental import pallas as pl
from jax.experimental.pallas import tpu as pltpu
from jax.experimental.pallas import tpu_sc as plsc

_B = 16384
_D = 64
_NC = 2
_NS = 16
_NW = _NC * _NS
_BPW = _B // _NW
_CHUNK = 128
_PASS = 256


def _sc_gather(table_padded, idx2d):
    mesh = plsc.VectorSubcoreMesh(core_axis_name="c", subcore_axis_name="s")

    @pl.kernel(
        out_type=jax.ShapeDtypeStruct((_B, 2 * _D), jnp.float32),
        mesh=mesh,
        scratch_types=[
            pltpu.VMEM((_BPW // _CHUNK, _CHUNK), jnp.int32),
            pltpu.VMEM((_PASS, 2 * _D), jnp.float32),
            pltpu.SemaphoreType.DMA,
        ],
    )
    def k(tab_hbm, idx_hbm, out_hbm, idx_v, rows, sem):
        wid = lax.axis_index("s") * _NC + lax.axis_index("c")
        base = wid * _BPW
        nidx = _BPW // _CHUNK
        pltpu.sync_copy(idx_hbm.at[pl.ds(wid * nidx, nidx)], idx_v)
        for p in range(_BPW // _PASS):
            copies = []
            for j in range(_PASS // _CHUNK):
                copies.append(pltpu.async_copy(
                    tab_hbm.at[idx_v.at[p * (_PASS // _CHUNK) + j]],
                    rows.at[pl.ds(j * _CHUNK, _CHUNK)], sem))
            for c in copies:
                c.wait()
            pltpu.sync_copy(rows, out_hbm.at[pl.ds(base + p * _PASS, _PASS)])

    return k(table_padded, idx2d)


def _pad_body(in_ref, out_ref):
    out_ref[:, :_D] = in_ref[...]


def _tc_pad(table, blk=8192):
    n = table.shape[0]
    grid = (pl.cdiv(n, blk),)
    return pl.pallas_call(
        _pad_body,
        out_shape=jax.ShapeDtypeStruct((n, 2 * _D), jnp.float32),
        grid=grid,
        in_specs=[pl.BlockSpec((blk, _D), lambda i: (i, 0))],
        out_specs=pl.BlockSpec((blk, 2 * _D), lambda i: (i, 0)),
        compiler_params=pltpu.CompilerParams(
            dimension_semantics=("parallel",)),
    )(table)


def _mlp_body(item_ref, text_ref, w1_ref, b1_ref, w2_ref, b2_ref, out_ref):
    item = item_ref[:, :_D]
    text = text_ref[:, :_D]
    w1 = w1_ref[...]
    cdims = (((1,), (1,)), ((), ()))
    h = lax.dot_general(item, w1[:, :_D], cdims,
                        preferred_element_type=jnp.float32)
    h = h + lax.dot_general(text, w1[:, _D:], cdims,
                            preferred_element_type=jnp.float32)
    h = jnp.maximum(h + b1_ref[...], 0.0)
    w2 = w2_ref[...]
    logits = lax.dot_general(h, w2, cdims, preferred_element_type=jnp.float32)
    b2v = b2_ref[...]
    d = (logits[:, 0:1] - logits[:, 1:2]) + (b2v[0, 0] - b2v[0, 1])
    g0 = 1.0 / (1.0 + jnp.exp(-d))
    out_ref[...] = text + g0 * (item - text)


def _tc_gating(item_emb, text_emb, W1, b1, W2, b2, blk=4096):
    grid = (_B // blk,)
    return pl.pallas_call(
        _mlp_body,
        out_shape=jax.ShapeDtypeStruct((_B, _D), jnp.float32),
        grid=grid,
        in_specs=[
            pl.BlockSpec((blk, 2 * _D), lambda i: (i, 0)),
            pl.BlockSpec((blk, 2 * _D), lambda i: (i, 0)),
            pl.BlockSpec((128, 128), lambda i: (0, 0)),
            pl.BlockSpec((1, 128), lambda i: (0, 0)),
            pl.BlockSpec((2, 128), lambda i: (0, 0)),
            pl.BlockSpec((1, 2), lambda i: (0, 0)),
        ],
        out_specs=pl.BlockSpec((blk, _D), lambda i: (i, 0)),
    )(item_emb, text_emb, W1, b1, W2, b2)


def kernel(item_ids, item_table, text_table, W1, b1, W2, b2):
    idx2d = item_ids.astype(jnp.int32).reshape(_B // _CHUNK, _CHUNK)
    item_pad = _tc_pad(item_table)
    text_pad = _tc_pad(text_table)
    item_rows = _sc_gather(item_pad, idx2d)
    text_rows = _sc_gather(text_pad, idx2d)
    return _tc_gating(item_rows, text_rows, W1,
                      b1.reshape(1, 128), W2, b2.reshape(1, 2))


# speedup vs baseline: 1.5372x; 1.5372x over previous
import jax
import jax.numpy as jnp
from jax import lax
from jax.experimental import pallas as pl
from jax.experimental.pallas import tpu as pltpu
from jax.experimental.pallas import tpu_sc as plsc

_B = 16384
_D = 64
_NC = 2
_NS = 16
_NW = _NC * _NS
_B_SC = 11264
_B_TC = _B - _B_SC
_BPW = _B_SC // _NW
_PASS = _BPW


def _sc_dual_gather(item_table, text_table, item_ids):
    mesh = plsc.VectorSubcoreMesh(core_axis_name="c", subcore_axis_name="s")
    out_t = (
        jax.ShapeDtypeStruct((_B_SC, _D), jnp.float32),
        jax.ShapeDtypeStruct((_B_SC, _D), jnp.float32),
    )

    @pl.kernel(
        out_type=out_t,
        mesh=mesh,
        scratch_types=[
            pltpu.VMEM((_BPW,), jnp.int32),
            pltpu.VMEM((_PASS, _D), jnp.float32),
            pltpu.VMEM((_PASS, _D), jnp.float32),
            pltpu.SemaphoreType.DMA,
            pltpu.SemaphoreType.DMA,
        ],
    )
    def k(item_hbm, text_hbm, idx_hbm, oi_hbm, ot_hbm, idx_v,
          irows, trows, sem, sem2):
        wid = lax.axis_index("s") * _NC + lax.axis_index("c")
        base = wid * _BPW
        pltpu.sync_copy(idx_hbm.at[pl.ds(base, _BPW)], idx_v)
        for p in range(_BPW // _PASS):
            @plsc.parallel_loop(0, _PASS, step=16)
            def _(j):
                v = idx_v[pl.ds(p * _PASS + j, 16)]
                for q in range(16):
                    r = v[q]
                    pltpu.async_copy(item_hbm.at[pl.ds(r, 1)],
                                     irows.at[pl.ds(j + q, 1)], sem)
                    pltpu.async_copy(text_hbm.at[pl.ds(r, 1)],
                                     trows.at[pl.ds(j + q, 1)], sem2)
            pltpu.make_async_copy(item_hbm.at[pl.ds(0, _PASS)], irows, sem).wait()
            pltpu.make_async_copy(text_hbm.at[pl.ds(0, _PASS)], trows, sem2).wait()
            pltpu.sync_copy(irows, oi_hbm.at[pl.ds(base + p * _PASS, _PASS)])
            pltpu.sync_copy(trows, ot_hbm.at[pl.ds(base + p * _PASS, _PASS)])

    return k(item_table, text_table, item_ids)


def _tc_gather(item_table, text_table, ids_tail):
    h = ids_tail.shape[0]
    hc = h // 2

    def body(ids_ref, item_any, text_any, oi_any, ot_any, sem_i, sem_t):
        g = pl.program_id(0)
        base = g * hc

        def loop(j, c):
            r = ids_ref[base + j]
            pltpu.async_copy(item_any.at[pl.ds(r, 1)],
                             oi_any.at[pl.ds(base + j, 1)], sem_i)
            pltpu.async_copy(text_any.at[pl.ds(r, 1)],
                             ot_any.at[pl.ds(base + j, 1)], sem_t)
            return c

        lax.fori_loop(0, hc, loop, 0, unroll=8)
        pltpu.make_async_copy(item_any.at[pl.ds(0, hc)],
                              oi_any.at[pl.ds(base, hc)], sem_i).wait()
        pltpu.make_async_copy(text_any.at[pl.ds(0, hc)],
                              ot_any.at[pl.ds(base, hc)], sem_t).wait()

    grid_spec = pltpu.PrefetchScalarGridSpec(
        num_scalar_prefetch=1,
        grid=(2,),
        in_specs=[pl.BlockSpec(memory_space=pl.ANY),
                  pl.BlockSpec(memory_space=pl.ANY)],
        out_specs=[pl.BlockSpec(memory_space=pl.ANY),
                   pl.BlockSpec(memory_space=pl.ANY)],
        scratch_shapes=[pltpu.SemaphoreType.DMA, pltpu.SemaphoreType.DMA],
    )
    return pl.pallas_call(
        body,
        grid_spec=grid_spec,
        out_shape=(jax.ShapeDtypeStruct((h, _D), jnp.float32),
                   jax.ShapeDtypeStruct((h, _D), jnp.float32)),
        compiler_params=pltpu.CompilerParams(
            dimension_semantics=("parallel",)),
    )(ids_tail, item_table, text_table)


def _mlp_body(item_ref, text_ref, w1_ref, b1_ref, w2_ref, b2_ref, out_ref):
    item = item_ref[...]
    text = text_ref[...]
    w1 = w1_ref[...]
    cdims = (((1,), (1,)), ((), ()))
    h = lax.dot_general(item, w1[:, :_D], cdims,
                        preferred_element_type=jnp.float32)
    h = h + lax.dot_general(text, w1[:, _D:], cdims,
                            preferred_element_type=jnp.float32)
    h = jnp.maximum(h + b1_ref[...], 0.0)
    w2 = w2_ref[...]
    logits = lax.dot_general(h, w2, cdims, preferred_element_type=jnp.float32)
    b2v = b2_ref[...]
    d = (logits[:, 0:1] - logits[:, 1:2]) + (b2v[0, 0] - b2v[0, 1])
    g0 = 1.0 / (1.0 + jnp.exp(-d))
    out_ref[...] = text + g0 * (item - text)


def _tc_gating(item_emb, text_emb, W1, b1, W2, b2, blk=4096):
    grid = (_B // blk,)
    return pl.pallas_call(
        _mlp_body,
        out_shape=jax.ShapeDtypeStruct((_B, _D), jnp.float32),
        grid=grid,
        in_specs=[
            pl.BlockSpec((blk, _D), lambda i: (i, 0)),
            pl.BlockSpec((blk, _D), lambda i: (i, 0)),
            pl.BlockSpec((128, 128), lambda i: (0, 0)),
            pl.BlockSpec((1, 128), lambda i: (0, 0)),
            pl.BlockSpec((2, 128), lambda i: (0, 0)),
            pl.BlockSpec((1, 2), lambda i: (0, 0)),
        ],
        out_specs=pl.BlockSpec((blk, _D), lambda i: (i, 0)),
    )(item_emb, text_emb, W1, b1, W2, b2)


def kernel(item_ids, item_table, text_table, W1, b1, W2, b2):
    ids32 = item_ids.astype(jnp.int32)
    oi_tc, ot_tc = _tc_gather(item_table, text_table, ids32[_B_SC:])
    oi_sc, ot_sc = _sc_dual_gather(item_table, text_table, ids32[:_B_SC])
    item_emb = jnp.concatenate([oi_sc, oi_tc], axis=0)
    text_emb = jnp.concatenate([ot_sc, ot_tc], axis=0)
    return _tc_gating(item_emb, text_emb, W1,
                      b1.reshape(1, 128), W2, b2.reshape(1, 2))


# speedup vs baseline: 1.8781x; 1.2218x over previous
import jax
import jax.numpy as jnp
from jax import lax
from jax.experimental import pallas as pl
from jax.experimental.pallas import tpu as pltpu
from jax.experimental.pallas import tpu_sc as plsc

_B = 16384
_D = 64
_NC = 2
_NS = 16
_NW = _NC * _NS
_BPW = _B // _NW
_PASS = 256


def _sc_dual_gather(item_table, text_table, item_ids):
    mesh = plsc.VectorSubcoreMesh(core_axis_name="c", subcore_axis_name="s")
    out_t = (
        jax.ShapeDtypeStruct((_B, _D), jnp.float32),
        jax.ShapeDtypeStruct((_B, _D), jnp.float32),
    )

    @pl.kernel(
        out_type=out_t,
        mesh=mesh,
        scratch_types=[
            pltpu.VMEM((_BPW,), jnp.int32),
            pltpu.VMEM((_PASS, _D), jnp.float32),
            pltpu.VMEM((_PASS, _D), jnp.float32),
            pltpu.SemaphoreType.DMA,
            pltpu.SemaphoreType.DMA,
        ],
    )
    def k(item_hbm, text_hbm, idx_hbm, oi_hbm, ot_hbm, idx_v,
          irows, trows, sem, sem2):
        wid = lax.axis_index("s") * _NC + lax.axis_index("c")
        base = wid * _BPW
        pltpu.sync_copy(idx_hbm.at[pl.ds(base, _BPW)], idx_v)
        for p in range(_BPW // _PASS):
            @plsc.parallel_loop(0, _PASS, step=16)
            def _(j):
                v = idx_v[pl.ds(p * _PASS + j, 16)]
                for q in range(16):
                    r = v[q]
                    pltpu.async_copy(item_hbm.at[pl.ds(r, 1)],
                                     irows.at[pl.ds(j + q, 1)], sem)
                    pltpu.async_copy(text_hbm.at[pl.ds(r, 1)],
                                     trows.at[pl.ds(j + q, 1)], sem2)
            pltpu.make_async_copy(item_hbm.at[pl.ds(0, _PASS)], irows, sem).wait()
            pltpu.make_async_copy(text_hbm.at[pl.ds(0, _PASS)], trows, sem2).wait()
            pltpu.sync_copy(irows, oi_hbm.at[pl.ds(base + p * _PASS, _PASS)])
            pltpu.sync_copy(trows, ot_hbm.at[pl.ds(base + p * _PASS, _PASS)])

    return k(item_table, text_table, item_ids)


def _mlp_body(item_ref, text_ref, w1_ref, b1_ref, w2_ref, b2_ref, out_ref):
    item = item_ref[...]
    text = text_ref[...]
    w1 = w1_ref[...]
    cdims = (((1,), (1,)), ((), ()))
    h = lax.dot_general(item, w1[:, :_D], cdims,
                        preferred_element_type=jnp.float32)
    h = h + lax.dot_general(text, w1[:, _D:], cdims,
                            preferred_element_type=jnp.float32)
    h = jnp.maximum(h + b1_ref[...], 0.0)
    w2 = w2_ref[...]
    logits = lax.dot_general(h, w2, cdims, preferred_element_type=jnp.float32)
    b2v = b2_ref[...]
    d = (logits[:, 0:1] - logits[:, 1:2]) + (b2v[0, 0] - b2v[0, 1])
    g0 = 1.0 / (1.0 + jnp.exp(-d))
    out_ref[...] = text + g0 * (item - text)


def _tc_gating(item_emb, text_emb, W1, b1, W2, b2, blk=4096):
    grid = (_B // blk,)
    return pl.pallas_call(
        _mlp_body,
        out_shape=jax.ShapeDtypeStruct((_B, _D), jnp.float32),
        grid=grid,
        in_specs=[
            pl.BlockSpec((blk, _D), lambda i: (i, 0)),
            pl.BlockSpec((blk, _D), lambda i: (i, 0)),
            pl.BlockSpec((128, 128), lambda i: (0, 0)),
            pl.BlockSpec((1, 128), lambda i: (0, 0)),
            pl.BlockSpec((2, 128), lambda i: (0, 0)),
            pl.BlockSpec((1, 2), lambda i: (0, 0)),
        ],
        out_specs=pl.BlockSpec((blk, _D), lambda i: (i, 0)),
    )(item_emb, text_emb, W1, b1, W2, b2)


def kernel(item_ids, item_table, text_table, W1, b1, W2, b2):
    item_emb, text_emb = _sc_dual_gather(
        item_table, text_table, item_ids.astype(jnp.int32))
    return _tc_gating(item_emb, text_emb, W1,
                      b1.reshape(1, 128), W2, b2.reshape(1, 2))
